# baseline (speedup 1.0000x reference)
import functools

import jax
import jax.numpy as jnp
from jax import lax
from jax.experimental import pallas as pl
from jax.experimental.pallas import tpu as pltpu
from jax.experimental.pallas import tpu_sc as plsc

F32 = jnp.float32
NC = 2
NS = 16
NW = NC * NS
K = 80
S = 5
SC2_GROUPS = ((0, 1), (2, 3, 4))

_mesh = plsc.VectorSubcoreMesh(core_axis_name="c", subcore_axis_name="s")


def _unpack_hi(p):
    w = lax.bitcast_convert_type(p, jnp.uint32)
    return lax.bitcast_convert_type((w >> 16).astype(jnp.uint16),
                                    jnp.bfloat16).astype(F32)


def _unpack_lo(p):
    w = lax.bitcast_convert_type(p, jnp.uint32)
    return lax.bitcast_convert_type(w.astype(jnp.uint16),
                                    jnp.bfloat16).astype(F32)


def _node_matmuls(h, wts, bs, block=2000):
    N, D = h.shape
    nb = N // block

    def pack2(x, y):
        xu = lax.bitcast_convert_type(x.astype(jnp.bfloat16),
                                      jnp.uint16).astype(jnp.uint32)
        yu = lax.bitcast_convert_type(y.astype(jnp.bfloat16),
                                      jnp.uint16).astype(jnp.uint32)
        return lax.bitcast_convert_type((xu << 16) | yu, F32)

    def body(h_ref, w_ref, b_ref, oa1, ot1, ot2):
        hh = h_ref[...]
        m = [jnp.dot(hh, w_ref[i], precision=lax.Precision.HIGHEST,
                     preferred_element_type=F32) + b_ref[i] for i in range(5)]
        oa1[...] = m[0]
        ot1[...] = pack2(m[1], m[2])
        ot2[...] = pack2(m[3], m[4])

    blk = pl.BlockSpec((block, D), lambda i: (i, 0))
    return pl.pallas_call(
        body,
        grid=(nb,),
        in_specs=[blk,
                  pl.BlockSpec((5, D, D), lambda i: (0, 0, 0)),
                  pl.BlockSpec((5, 1, D), lambda i: (0, 0, 0))],
        out_specs=[blk, blk, blk],
        out_shape=[jax.ShapeDtypeStruct((N, D), F32)] * 3,
    )(h, jnp.stack(wts), jnp.stack(bs)[:, None, :])


def _sc_gather(ei_s, T1, T2, Es):
    N, D = T1.shape
    nchunk = Es // (NW * K)
    npair = (nchunk - 1) // 2
    fo = jax.ShapeDtypeStruct((Es, D), F32)

    @functools.partial(
        pl.kernel, mesh=_mesh,
        out_type=[fo, fo],
        scratch_types=[
            pltpu.VMEM((nchunk, K), jnp.int32),
            pltpu.VMEM((nchunk, K), jnp.int32),
            pltpu.VMEM((K, D), F32),
            pltpu.VMEM((K, D), F32),
            pltpu.VMEM((K, D), F32),
            pltpu.VMEM((K, D), F32),
            pltpu.SemaphoreType.DMA,
            pltpu.SemaphoreType.DMA,
            pltpu.SemaphoreType.DMA,
            pltpu.SemaphoreType.DMA,
        ],
    )
    def k(ei_hbm, t1_hbm, t2_hbm, g1_hbm, g2_hbm,
          idxs2, idxd2, a0, b0, a1, b1, sg0, sg1, sw0, sw1):
        wid = lax.axis_index("s") * NC + lax.axis_index("c")
        base = wid * nchunk
        pltpu.sync_copy(ei_hbm.at[0, wid], idxs2)
        pltpu.sync_copy(ei_hbm.at[1, wid], idxd2)

        def gath(i, bufa, bufb, sem):
            pltpu.async_copy(t1_hbm.at[idxs2.at[i]], bufa, sem)
            pltpu.async_copy(t2_hbm.at[idxd2.at[i]], bufb, sem)

        def wait_g(bufa, bufb, sem):
            pltpu.make_async_copy(g1_hbm.at[pl.ds(0, K)], bufa, sem).wait()
            pltpu.make_async_copy(g1_hbm.at[pl.ds(0, K)], bufb, sem).wait()

        def wout(i, bufa, bufb, sem):
            off = (base + i) * K
            pltpu.async_copy(bufa, g1_hbm.at[pl.ds(off, K)], sem)
            pltpu.async_copy(bufb, g2_hbm.at[pl.ds(off, K)], sem)

        def wait_w(bufa, bufb, sem):
            pltpu.make_async_copy(g1_hbm.at[pl.ds(0, K)], bufa, sem).wait()
            pltpu.make_async_copy(g1_hbm.at[pl.ds(0, K)], bufb, sem).wait()

        gath(0, a0, b0, sg0)
        gath(1, a1, b1, sg1)

        @pl.loop(0, npair)
        def _(j):
            i = j * 2
            wait_g(a0, b0, sg0)
            wout(i, a0, b0, sw0)
            wait_g(a1, b1, sg1)
            wout(i + 1, a1, b1, sw1)
            wait_w(a0, b0, sw0)
            gath(i + 2, a0, b0, sg0)
            wait_w(a1, b1, sw1)

            @pl.when(j < npair - 1)
            def _():
                gath(i + 3, a1, b1, sg1)

        wait_g(a0, b0, sg0)
        wout(nchunk - 1, a0, b0, sw0)
        wait_w(a0, b0, sw0)

    return k(ei_s, T1, T2)


def _edge_t_stats(e, G1, G2, wt, b, Es, eoff, block):
    D = e.shape[1]
    nb = Es // block
    ob = eoff // block

    def body(e_ref, g1_ref, g2_ref, w_ref, b_ref, st_ref, acc):
        i = pl.program_id(0)

        @pl.when(i == 0)
        def _():
            acc[...] = jnp.zeros_like(acc)

        t = (jnp.dot(e_ref[...], w_ref[...], precision=lax.Precision.HIGHEST,
                     preferred_element_type=F32)
             + b_ref[...] + _unpack_hi(g1_ref[...])
             + _unpack_hi(g2_ref[...]))
        t3 = t.reshape(block // 8, 8, D)
        acc[0] += jnp.sum(t3, axis=0)
        acc[1] += jnp.sum(t3 * t3, axis=0)

        @pl.when(i == nb - 1)
        def _():
            st_ref[...] = acc[...]

    blk = pl.BlockSpec((block, D), lambda i: (i, 0))
    eblk = pl.BlockSpec((block, D), lambda i: (ob + i, 0))
    return pl.pallas_call(
        body,
        grid=(nb,),
        in_specs=[eblk, blk, blk,
                  pl.BlockSpec((D, D), lambda i: (0, 0)),
                  pl.BlockSpec((1, D), lambda i: (0, 0))],
        out_specs=pl.BlockSpec((2, 8, D), lambda i: (0, 0, 0)),
        out_shape=jax.ShapeDtypeStruct((2, 8, D), F32),
        scratch_shapes=[pltpu.VMEM((2, 8, D), F32)],
    )(e, G1, G2, wt, b[None, :])


def _edge_sigma(e, G1, G2, wt, b, stats, gamma, beta, E_total, Es, eoff,
                block):
    D = e.shape[1]
    nb = Es // block
    ob = eoff // block
    ns = stats.shape[0]

    def body(e_ref, a2_ref, a3_ref, w_ref, b_ref, st_ref, g_ref, bt_ref,
             pf_ref, pb_ref):
        ssum = jnp.sum(st_ref[:, 0], axis=(0, 1))
        ssq = jnp.sum(st_ref[:, 1], axis=(0, 1))
        mean = ssum / E_total
        var = ssq / E_total - mean * mean
        scale = g_ref[0] * lax.rsqrt(var + 1e-5)
        shift = bt_ref[0] - mean * scale
        ew = e_ref[...]
        t = (jnp.dot(ew, w_ref[...], precision=lax.Precision.HIGHEST,
                     preferred_element_type=F32)
             + b_ref[...] + _unpack_hi(a2_ref[...])
             + _unpack_hi(a3_ref[...]))
        bn = t * scale + shift
        ee = jnp.maximum(bn, 0.0) + ew
        sg = jax.nn.sigmoid(ee)
        u2 = sg * _unpack_lo(a2_ref[...])
        u3 = sg * _unpack_lo(a3_ref[...])

        def planes(u):
            p0 = jnp.concatenate([u[:, 0:32], sg[:, 0:32],
                                  u[:, 32:64], sg[:, 32:64]], axis=1)
            p1 = jnp.concatenate([u[:, 64:96], sg[:, 64:96],
                                  u[:, 96:128], sg[:, 96:128]], axis=1)
            return jnp.stack([p0, p1], axis=0)

        pf_ref[...] = planes(u2)
        pb_ref[...] = planes(u3)

    blk = pl.BlockSpec((block, D), lambda i: (i, 0))
    eblk = pl.BlockSpec((block, D), lambda i: (ob + i, 0))
    small = pl.BlockSpec((1, D), lambda i: (0, 0))
    return pl.pallas_call(
        body,
        grid=(nb,),
        in_specs=[eblk, blk, blk,
                  pl.BlockSpec((D, D), lambda i: (0, 0)), small,
                  pl.BlockSpec((ns, 2, 8, D), lambda i: (0, 0, 0, 0)),
                  small, small],
        out_specs=[pl.BlockSpec((2, block, D), lambda i: (0, i, 0))] * 2,
        out_shape=[jax.ShapeDtypeStruct((2, Es, D), F32)] * 2,
    )(e, G1, G2, wt, b[None, :], stats, gamma[None, :], beta[None, :])


def _sc_scatter_group(ei_list, P_list, zrows, Np, idx_row):
    D = P_list[0].shape[2]
    Es = P_list[0].shape[1]
    nchunk = Es // (NW * K)
    npair = (nchunk - 1) // 2
    nrow = Np // NS
    ng = len(P_list)
    po = jax.ShapeDtypeStruct((NC, 2, Np, D), F32)

    @functools.partial(
        pl.kernel, mesh=_mesh,
        out_type=po,
        scratch_types=[
            pltpu.VMEM((nchunk, K), jnp.int32),
            pltpu.VMEM((K, D), F32),
            pltpu.VMEM((K, D), F32),
            pltpu.VMEM_SHARED((Np, D), F32),
            pltpu.SemaphoreType.DMA,
            pltpu.SemaphoreType.DMA,
            pltpu.SemaphoreType.DMA,
            pltpu.SemaphoreType.DMA,
        ],
    )
    def k(*refs):
        ei_hbms = refs[:ng]
        p_hbms = refs[ng:2 * ng]
        z_hbm = refs[2 * ng]
        out = refs[2 * ng + 1]
        idx2, f0, f1, acc, sr0, sr1, ss0, ss1 = refs[2 * ng + 2:]
        cid = lax.axis_index("c")
        sid = lax.axis_index("s")
        wid = sid * NC + cid
        base = wid * nchunk
        r0 = sid * nrow
        for p in range(2):
            pltpu.sync_copy(z_hbm, acc.at[pl.ds(r0, nrow)])
            plsc.subcore_barrier()
            for g in range(ng):
                p_hbm = p_hbms[g]
                pltpu.sync_copy(ei_hbms[g].at[idx_row, wid], idx2)

                def rd(i, buf, sem):
                    off = (base + i) * K
                    pltpu.async_copy(p_hbm.at[p, pl.ds(off, K), :], buf, sem)

                def wait_r(buf, sem):
                    pltpu.make_async_copy(
                        p_hbm.at[0, pl.ds(0, K), :], buf, sem).wait()

                def scat(i, buf, sem):
                    pltpu.async_copy(buf, acc.at[idx2.at[i]], sem, add=True)

                def wait_s(buf, sem):
                    pltpu.make_async_copy(
                        p_hbm.at[0, pl.ds(0, K), :], buf, sem).wait()

                rd(0, f0, sr0)
                rd(1, f1, sr1)

                @pl.loop(0, npair)
                def _(j):
                    i = j * 2
                    wait_r(f0, sr0)
                    scat(i, f0, ss0)
                    wait_r(f1, sr1)
                    scat(i + 1, f1, ss1)
                    wait_s(f0, ss0)
                    rd(i + 2, f0, sr0)
                    wait_s(f1, ss1)

                    @pl.when(j < npair - 1)
                    def _():
                        rd(i + 3, f1, sr1)

                wait_r(f0, sr0)
                scat(nchunk - 1, f0, ss0)
                wait_s(f0, ss0)

            plsc.subcore_barrier()
            pltpu.sync_copy(acc.at[pl.ds(r0, nrow)],
                            out.at[cid, p, pl.ds(r0, nrow), :])
            plsc.subcore_barrier()

    return k(*ei_list, *P_list, zrows)


def _final(A1h, h, oFs, oBs, gamma, beta, block):
    N, D = h.shape
    nb = N // block
    nparts = len(oFs)

    def body(*refs):
        a1_ref, h_ref = refs[0], refs[1]
        f_refs = refs[2:2 + nparts]
        b_refs = refs[2 + nparts:2 + 2 * nparts]
        g_ref, bt_ref, o_ref, acc, hp_ref = refs[2 + 2 * nparts:]
        p = pl.program_id(0)
        j = pl.program_id(1)

        def agg(part_refs):
            cols = []
            for c in range(4):
                pp = c // 2
                a = slice(64 * (c % 2), 64 * (c % 2) + 32)
                d = slice(64 * (c % 2) + 32, 64 * (c % 2) + 64)
                num = den = None
                for r in part_refs:
                    for cc in range(2):
                        nn = r[cc, pp, :, a]
                        dd = r[cc, pp, :, d]
                        num = nn if num is None else num + nn
                        den = dd if den is None else den + dd
                cols.append(num / (den + 1e-6))
            return jnp.concatenate(cols, axis=1)

        @pl.when(p == 0)
        def _():
            hp = a1_ref[...] + agg(f_refs) + agg(b_refs)
            hp_ref[pl.ds(j * block, block), :] = hp

            @pl.when(j == 0)
            def _():
                acc[...] = jnp.zeros_like(acc)

            h3 = hp.reshape(block // 8, 8, D)
            acc[0] += jnp.sum(h3, axis=0)
            acc[1] += jnp.sum(h3 * h3, axis=0)

        @pl.when(p == 1)
        def _():
            hp = hp_ref[pl.ds(j * block, block), :]
            mean = jnp.sum(acc[0], axis=0) / N
            var = jnp.sum(acc[1], axis=0) / N - mean * mean
            scale = g_ref[0] * lax.rsqrt(var + 1e-5)
            shift = bt_ref[0] - mean * scale
            o_ref[...] = jnp.maximum(hp * scale + shift, 0.0) + h_ref[...]

    blk = pl.BlockSpec((block, D), lambda p, j: (j, 0))
    pblk = pl.BlockSpec((2, 2, block, D),
                        lambda p, j: (0, 0, j * (1 - p), 0))
    small = pl.BlockSpec((1, D), lambda p, j: (0, 0))
    return pl.pallas_call(
        body,
        grid=(2, nb),
        in_specs=[blk, blk] + [pblk] * (2 * nparts) + [small, small],
        out_specs=blk,
        out_shape=jax.ShapeDtypeStruct((N, D), F32),
        scratch_shapes=[pltpu.VMEM((2, 8, D), F32),
                        pltpu.VMEM((N, D), F32)],
    )(A1h, h, *oFs, *oBs, gamma[None, :], beta[None, :])


def kernel(edge_index, h, e, A1_W, A1_b, A2_W, A2_b, A3_W, A3_b,
           B1_W, B1_b, B2_W, B2_b, B3_W, B3_b,
           bn_h_gamma, bn_h_beta, bn_e_gamma, bn_e_beta):
    N, D = h.shape
    E = e.shape[0]
    Es = E // S
    nchunk = Es // (NW * K)

    A1h, T1, T2 = _node_matmuls(
        h,
        (A1_W.T, B1_W.T, A2_W.T, B2_W.T, A3_W.T),
        (A1_b, B1_b, A2_b, B2_b, A3_b))

    ei6 = edge_index.reshape(2, S, NW, nchunk, K)
    ei_s = [ei6[:, s] for s in range(S)]

    G = [_sc_gather(ei_s[s], T1, T2, Es) for s in range(S)]
    stats = jnp.stack(
        [_edge_t_stats(e, G[s][0], G[s][1], B3_W.T, B3_b, Es, s * Es, 2000)
         for s in range(S)])

    P = [_edge_sigma(e, G[s][0], G[s][1], B3_W.T, B3_b, stats,
                     bn_e_gamma, bn_e_beta, E, Es, s * Es, 2000)
         for s in range(S)]

    Np = ((N + 8 * NS - 1) // (8 * NS)) * (8 * NS)
    zrows = jnp.zeros((Np // NS, D), F32)
    oFs, oBs = [], []
    for s in range(S):
        oFs.append(_sc_scatter_group([ei_s[s]], [P[s][0]], zrows, Np, 1))
        oBs.append(_sc_scatter_group([ei_s[s]], [P[s][1]], zrows, Np, 0))

    return _final(A1h, h, oFs, oBs, bn_h_gamma, bn_h_beta, 1000)


# speedup vs baseline: 1.1234x; 1.1234x over previous
import functools

import jax
import jax.numpy as jnp
from jax import lax
from jax.experimental import pallas as pl
from jax.experimental.pallas import tpu as pltpu
from jax.experimental.pallas import tpu_sc as plsc

F32 = jnp.float32
NC = 2
NS = 16
NW = NC * NS
K = 80
S = 5
SC2_GROUPS = ((0, 1), (2, 3, 4))

_mesh = plsc.VectorSubcoreMesh(core_axis_name="c", subcore_axis_name="s")


def _unpack_hi(p):
    w = lax.bitcast_convert_type(p, jnp.uint32)
    return lax.bitcast_convert_type((w >> 16).astype(jnp.uint16),
                                    jnp.bfloat16).astype(F32)


def _unpack_lo(p):
    w = lax.bitcast_convert_type(p, jnp.uint32)
    return lax.bitcast_convert_type(w.astype(jnp.uint16),
                                    jnp.bfloat16).astype(F32)


def _node_matmuls(h, wts, bs, block=2000):
    N, D = h.shape
    nb = N // block

    def pack2(x, y):
        xu = lax.bitcast_convert_type(x.astype(jnp.bfloat16),
                                      jnp.uint16).astype(jnp.uint32)
        yu = lax.bitcast_convert_type(y.astype(jnp.bfloat16),
                                      jnp.uint16).astype(jnp.uint32)
        return lax.bitcast_convert_type((xu << 16) | yu, F32)

    def body(h_ref, w_ref, b_ref, oa1, ot1, ot2):
        hh = h_ref[...]
        m = [jnp.dot(hh, w_ref[i], precision=lax.Precision.HIGHEST,
                     preferred_element_type=F32) + b_ref[i] for i in range(5)]
        oa1[...] = m[0]
        ot1[...] = pack2(m[1], m[2])
        ot2[...] = pack2(m[3], m[4])

    blk = pl.BlockSpec((block, D), lambda i: (i, 0))
    return pl.pallas_call(
        body,
        grid=(nb,),
        in_specs=[blk,
                  pl.BlockSpec((5, D, D), lambda i: (0, 0, 0)),
                  pl.BlockSpec((5, 1, D), lambda i: (0, 0, 0))],
        out_specs=[blk, blk, blk],
        out_shape=[jax.ShapeDtypeStruct((N, D), F32)] * 3,
    )(h, jnp.stack(wts), jnp.stack(bs)[:, None, :])


def _sc_gather(ei_s, T1, T2, Es):
    N, D = T1.shape
    nchunk = Es // (NW * K)
    npair = (nchunk - 1) // 2
    fo = jax.ShapeDtypeStruct((Es, D), F32)

    @functools.partial(
        pl.kernel, mesh=_mesh,
        out_type=[fo, fo],
        scratch_types=[
            pltpu.VMEM((nchunk, K), jnp.int32),
            pltpu.VMEM((nchunk, K), jnp.int32),
            pltpu.VMEM((K, D), F32),
            pltpu.VMEM((K, D), F32),
            pltpu.VMEM((K, D), F32),
            pltpu.VMEM((K, D), F32),
            pltpu.SemaphoreType.DMA,
            pltpu.SemaphoreType.DMA,
            pltpu.SemaphoreType.DMA,
            pltpu.SemaphoreType.DMA,
        ],
    )
    def k(ei_hbm, t1_hbm, t2_hbm, g1_hbm, g2_hbm,
          idxs2, idxd2, a0, b0, a1, b1, sg0, sg1, sw0, sw1):
        wid = lax.axis_index("s") * NC + lax.axis_index("c")
        base = wid * nchunk
        pltpu.sync_copy(ei_hbm.at[0, wid], idxs2)
        pltpu.sync_copy(ei_hbm.at[1, wid], idxd2)

        def gath(i, bufa, bufb, sem):
            pltpu.async_copy(t1_hbm.at[idxs2.at[i]], bufa, sem)
            pltpu.async_copy(t2_hbm.at[idxd2.at[i]], bufb, sem)

        def wait_g(bufa, bufb, sem):
            pltpu.make_async_copy(g1_hbm.at[pl.ds(0, K)], bufa, sem).wait()
            pltpu.make_async_copy(g1_hbm.at[pl.ds(0, K)], bufb, sem).wait()

        def wout(i, bufa, bufb):
            off = (base + i) * K
            c1 = pltpu.async_copy(bufa, g1_hbm.at[pl.ds(off, K)], sw0)
            c2 = pltpu.async_copy(bufb, g2_hbm.at[pl.ds(off, K)], sw0)
            c1.wait()
            c2.wait()

        gath(0, a0, b0, sg0)
        gath(1, a1, b1, sg1)

        @pl.loop(0, npair)
        def _(j):
            i = j * 2
            wait_g(a0, b0, sg0)
            wout(i, a0, b0)
            gath(i + 2, a0, b0, sg0)
            wait_g(a1, b1, sg1)
            wout(i + 1, a1, b1)

            @pl.when(j < npair - 1)
            def _():
                gath(i + 3, a1, b1, sg1)

        wait_g(a0, b0, sg0)
        wout(nchunk - 1, a0, b0)

    return k(ei_s, T1, T2)


def _edge_t_stats(e, G1, G2, wt, b, Es, eoff, block):
    D = e.shape[1]
    nb = Es // block
    ob = eoff // block

    def body(e_ref, g1_ref, g2_ref, w_ref, b_ref, st_ref, acc):
        i = pl.program_id(0)

        @pl.when(i == 0)
        def _():
            acc[...] = jnp.zeros_like(acc)

        t = (jnp.dot(e_ref[...], w_ref[...], precision=lax.Precision.HIGHEST,
                     preferred_element_type=F32)
             + b_ref[...] + _unpack_hi(g1_ref[...])
             + _unpack_hi(g2_ref[...]))
        t3 = t.reshape(block // 8, 8, D)
        acc[0] += jnp.sum(t3, axis=0)
        acc[1] += jnp.sum(t3 * t3, axis=0)

        @pl.when(i == nb - 1)
        def _():
            st_ref[...] = acc[...]

    blk = pl.BlockSpec((block, D), lambda i: (i, 0))
    eblk = pl.BlockSpec((block, D), lambda i: (ob + i, 0))
    return pl.pallas_call(
        body,
        grid=(nb,),
        in_specs=[eblk, blk, blk,
                  pl.BlockSpec((D, D), lambda i: (0, 0)),
                  pl.BlockSpec((1, D), lambda i: (0, 0))],
        out_specs=pl.BlockSpec((2, 8, D), lambda i: (0, 0, 0)),
        out_shape=jax.ShapeDtypeStruct((2, 8, D), F32),
        scratch_shapes=[pltpu.VMEM((2, 8, D), F32)],
    )(e, G1, G2, wt, b[None, :])


def _edge_sigma(e, G1, G2, wt, b, stats, gamma, beta, E_total, Es, eoff,
                block):
    D = e.shape[1]
    nb = Es // block
    ob = eoff // block
    ns = stats.shape[0]

    def body(e_ref, a2_ref, a3_ref, w_ref, b_ref, st_ref, g_ref, bt_ref,
             pf_ref, pb_ref):
        ssum = jnp.sum(st_ref[:, 0], axis=(0, 1))
        ssq = jnp.sum(st_ref[:, 1], axis=(0, 1))
        mean = ssum / E_total
        var = ssq / E_total - mean * mean
        scale = g_ref[0] * lax.rsqrt(var + 1e-5)
        shift = bt_ref[0] - mean * scale
        ew = e_ref[...]
        t = (jnp.dot(ew, w_ref[...], precision=lax.Precision.HIGHEST,
                     preferred_element_type=F32)
             + b_ref[...] + _unpack_hi(a2_ref[...])
             + _unpack_hi(a3_ref[...]))
        bn = t * scale + shift
        ee = jnp.maximum(bn, 0.0) + ew
        sg = jax.nn.sigmoid(ee)
        u2 = sg * _unpack_lo(a2_ref[...])
        u3 = sg * _unpack_lo(a3_ref[...])

        def planes(u):
            p0 = jnp.concatenate([u[:, 0:32], sg[:, 0:32],
                                  u[:, 32:64], sg[:, 32:64]], axis=1)
            p1 = jnp.concatenate([u[:, 64:96], sg[:, 64:96],
                                  u[:, 96:128], sg[:, 96:128]], axis=1)
            return jnp.stack([p0, p1], axis=0)

        pf_ref[...] = planes(u2)
        pb_ref[...] = planes(u3)

    blk = pl.BlockSpec((block, D), lambda i: (i, 0))
    eblk = pl.BlockSpec((block, D), lambda i: (ob + i, 0))
    small = pl.BlockSpec((1, D), lambda i: (0, 0))
    return pl.pallas_call(
        body,
        grid=(nb,),
        in_specs=[eblk, blk, blk,
                  pl.BlockSpec((D, D), lambda i: (0, 0)), small,
                  pl.BlockSpec((ns, 2, 8, D), lambda i: (0, 0, 0, 0)),
                  small, small],
        out_specs=[pl.BlockSpec((2, block, D), lambda i: (0, i, 0))] * 2,
        out_shape=[jax.ShapeDtypeStruct((2, Es, D), F32)] * 2,
    )(e, G1, G2, wt, b[None, :], stats, gamma[None, :], beta[None, :])


def _sc_scatter_group(ei_list, P_list, zrows, Np, idx_row):
    D = P_list[0].shape[2]
    Es = P_list[0].shape[1]
    nchunk = Es // (NW * K)
    npair = (nchunk - 1) // 2
    nrow = Np // NS
    ng = len(P_list)
    po = jax.ShapeDtypeStruct((NC, 2, Np, D), F32)

    @functools.partial(
        pl.kernel, mesh=_mesh,
        out_type=po,
        scratch_types=[
            pltpu.VMEM((nchunk, K), jnp.int32),
            pltpu.VMEM((K, D), F32),
            pltpu.VMEM((K, D), F32),
            pltpu.VMEM_SHARED((Np, D), F32),
            pltpu.SemaphoreType.DMA,
            pltpu.SemaphoreType.DMA,
            pltpu.SemaphoreType.DMA,
            pltpu.SemaphoreType.DMA,
        ],
    )
    def k(*refs):
        ei_hbms = refs[:ng]
        p_hbms = refs[ng:2 * ng]
        z_hbm = refs[2 * ng]
        out = refs[2 * ng + 1]
        idx2, f0, f1, acc, sr0, sr1, ss0, ss1 = refs[2 * ng + 2:]
        cid = lax.axis_index("c")
        sid = lax.axis_index("s")
        wid = sid * NC + cid
        base = wid * nchunk
        r0 = sid * nrow
        for p in range(2):
            pltpu.sync_copy(z_hbm, acc.at[pl.ds(r0, nrow)])
            plsc.subcore_barrier()
            for g in range(ng):
                p_hbm = p_hbms[g]
                pltpu.sync_copy(ei_hbms[g].at[idx_row, wid], idx2)

                def rd(i, buf, sem):
                    off = (base + i) * K
                    pltpu.async_copy(p_hbm.at[p, pl.ds(off, K), :], buf, sem)

                def wait_r(buf, sem):
                    pltpu.make_async_copy(
                        p_hbm.at[0, pl.ds(0, K), :], buf, sem).wait()

                def scat(i, buf):
                    pltpu.async_copy(buf, acc.at[idx2.at[i]], ss0,
                                     add=True).wait()

                rd(0, f0, sr0)
                rd(1, f1, sr1)

                @pl.loop(0, npair)
                def _(j):
                    i = j * 2
                    wait_r(f0, sr0)
                    scat(i, f0)
                    rd(i + 2, f0, sr0)
                    wait_r(f1, sr1)
                    scat(i + 1, f1)

                    @pl.when(j < npair - 1)
                    def _():
                        rd(i + 3, f1, sr1)

                wait_r(f0, sr0)
                scat(nchunk - 1, f0)

            plsc.subcore_barrier()
            pltpu.sync_copy(acc.at[pl.ds(r0, nrow)],
                            out.at[cid, p, pl.ds(r0, nrow), :])
            plsc.subcore_barrier()

    return k(*ei_list, *P_list, zrows)


def _final(A1h, h, oFs, oBs, gamma, beta, block):
    N, D = h.shape
    nb = N // block
    nparts = len(oFs)

    def body(*refs):
        a1_ref, h_ref = refs[0], refs[1]
        f_refs = refs[2:2 + nparts]
        b_refs = refs[2 + nparts:2 + 2 * nparts]
        g_ref, bt_ref, o_ref, acc, hp_ref = refs[2 + 2 * nparts:]
        p = pl.program_id(0)
        j = pl.program_id(1)

        def agg(part_refs):
            cols = []
            for c in range(4):
                pp = c // 2
                a = slice(64 * (c % 2), 64 * (c % 2) + 32)
                d = slice(64 * (c % 2) + 32, 64 * (c % 2) + 64)
                num = den = None
                for r in part_refs:
                    for cc in range(2):
                        nn = r[cc, pp, :, a]
                        dd = r[cc, pp, :, d]
                        num = nn if num is None else num + nn
                        den = dd if den is None else den + dd
                cols.append(num / (den + 1e-6))
            return jnp.concatenate(cols, axis=1)

        @pl.when(p == 0)
        def _():
            hp = a1_ref[...] + agg(f_refs) + agg(b_refs)
            hp_ref[pl.ds(j * block, block), :] = hp

            @pl.when(j == 0)
            def _():
                acc[...] = jnp.zeros_like(acc)

            h3 = hp.reshape(block // 8, 8, D)
            acc[0] += jnp.sum(h3, axis=0)
            acc[1] += jnp.sum(h3 * h3, axis=0)

        @pl.when(p == 1)
        def _():
            hp = hp_ref[pl.ds(j * block, block), :]
            mean = jnp.sum(acc[0], axis=0) / N
            var = jnp.sum(acc[1], axis=0) / N - mean * mean
            scale = g_ref[0] * lax.rsqrt(var + 1e-5)
            shift = bt_ref[0] - mean * scale
            o_ref[...] = jnp.maximum(hp * scale + shift, 0.0) + h_ref[...]

    blk = pl.BlockSpec((block, D), lambda p, j: (j, 0))
    pblk = pl.BlockSpec((2, 2, block, D),
                        lambda p, j: (0, 0, j * (1 - p), 0))
    small = pl.BlockSpec((1, D), lambda p, j: (0, 0))
    return pl.pallas_call(
        body,
        grid=(2, nb),
        in_specs=[blk, blk] + [pblk] * (2 * nparts) + [small, small],
        out_specs=blk,
        out_shape=jax.ShapeDtypeStruct((N, D), F32),
        scratch_shapes=[pltpu.VMEM((2, 8, D), F32),
                        pltpu.VMEM((N, D), F32)],
    )(A1h, h, *oFs, *oBs, gamma[None, :], beta[None, :])


def kernel(edge_index, h, e, A1_W, A1_b, A2_W, A2_b, A3_W, A3_b,
           B1_W, B1_b, B2_W, B2_b, B3_W, B3_b,
           bn_h_gamma, bn_h_beta, bn_e_gamma, bn_e_beta):
    N, D = h.shape
    E = e.shape[0]
    Es = E // S
    nchunk = Es // (NW * K)

    A1h, T1, T2 = _node_matmuls(
        h,
        (A1_W.T, B1_W.T, A2_W.T, B2_W.T, A3_W.T),
        (A1_b, B1_b, A2_b, B2_b, A3_b))

    ei6 = edge_index.reshape(2, S, NW, nchunk, K)
    ei_s = [ei6[:, s] for s in range(S)]

    G = [_sc_gather(ei_s[s], T1, T2, Es) for s in range(S)]
    stats = jnp.stack(
        [_edge_t_stats(e, G[s][0], G[s][1], B3_W.T, B3_b, Es, s * Es, 2000)
         for s in range(S)])

    P = [_edge_sigma(e, G[s][0], G[s][1], B3_W.T, B3_b, stats,
                     bn_e_gamma, bn_e_beta, E, Es, s * Es, 2000)
         for s in range(S)]

    Np = ((N + 8 * NS - 1) // (8 * NS)) * (8 * NS)
    zrows = jnp.zeros((Np // NS, D), F32)
    oFs, oBs = [], []
    for grp in ((0,), (1, 2), (3, 4)):
        eis = [ei_s[s] for s in grp]
        oFs.append(_sc_scatter_group(eis, [P[s][0] for s in grp],
                                     zrows, Np, 1))
        oBs.append(_sc_scatter_group(eis, [P[s][1] for s in grp],
                                     zrows, Np, 0))

    return _final(A1h, h, oFs, oBs, bn_h_gamma, bn_h_beta, 1000)


# speedup vs baseline: 1.1308x; 1.0066x over previous
import functools

import jax
import jax.numpy as jnp
from jax import lax
from jax.experimental import pallas as pl
from jax.experimental.pallas import tpu as pltpu
from jax.experimental.pallas import tpu_sc as plsc

F32 = jnp.float32
NC = 2
NS = 16
NW = NC * NS
K = 80
S = 5
SC2_GROUPS = ((0, 1), (2, 3, 4))

_mesh = plsc.VectorSubcoreMesh(core_axis_name="c", subcore_axis_name="s")


def _unpack_hi(p):
    w = lax.bitcast_convert_type(p, jnp.uint32)
    return lax.bitcast_convert_type((w >> 16).astype(jnp.uint16),
                                    jnp.bfloat16).astype(F32)


def _unpack_lo(p):
    w = lax.bitcast_convert_type(p, jnp.uint32)
    return lax.bitcast_convert_type(w.astype(jnp.uint16),
                                    jnp.bfloat16).astype(F32)


def _node_matmuls(h, wts, bs, block=2000):
    N, D = h.shape
    nb = N // block

    def pack2(x, y):
        xu = lax.bitcast_convert_type(x.astype(jnp.bfloat16),
                                      jnp.uint16).astype(jnp.uint32)
        yu = lax.bitcast_convert_type(y.astype(jnp.bfloat16),
                                      jnp.uint16).astype(jnp.uint32)
        return lax.bitcast_convert_type((xu << 16) | yu, F32)

    def body(h_ref, w_ref, b_ref, oa1, ot1, ot2):
        hh = h_ref[...]
        m = [jnp.dot(hh, w_ref[i], precision=lax.Precision.HIGHEST,
                     preferred_element_type=F32) + b_ref[i] for i in range(5)]
        oa1[...] = m[0]
        ot1[...] = pack2(m[1], m[2])
        ot2[...] = pack2(m[3], m[4])

    blk = pl.BlockSpec((block, D), lambda i: (i, 0))
    return pl.pallas_call(
        body,
        grid=(nb,),
        in_specs=[blk,
                  pl.BlockSpec((5, D, D), lambda i: (0, 0, 0)),
                  pl.BlockSpec((5, 1, D), lambda i: (0, 0, 0))],
        out_specs=[blk, blk, blk],
        out_shape=[jax.ShapeDtypeStruct((N, D), F32)] * 3,
    )(h, jnp.stack(wts), jnp.stack(bs)[:, None, :])


def _sc_gather(ei_s, T1, T2, Es):
    N, D = T1.shape
    nchunk = Es // (NW * K)
    npair = (nchunk - 1) // 2
    fo = jax.ShapeDtypeStruct((Es, D), F32)

    @functools.partial(
        pl.kernel, mesh=_mesh,
        out_type=[fo, fo],
        scratch_types=[
            pltpu.VMEM((nchunk, K), jnp.int32),
            pltpu.VMEM((nchunk, K), jnp.int32),
            pltpu.VMEM((K, D), F32),
            pltpu.VMEM((K, D), F32),
            pltpu.VMEM((K, D), F32),
            pltpu.VMEM((K, D), F32),
            pltpu.SemaphoreType.DMA,
            pltpu.SemaphoreType.DMA,
            pltpu.SemaphoreType.DMA,
            pltpu.SemaphoreType.DMA,
        ],
    )
    def k(ei_hbm, t1_hbm, t2_hbm, g1_hbm, g2_hbm,
          idxs2, idxd2, a0, b0, a1, b1, sg0, sg1, sw0, sw1):
        wid = lax.axis_index("s") * NC + lax.axis_index("c")
        base = wid * nchunk
        pltpu.sync_copy(ei_hbm.at[0, wid], idxs2)
        pltpu.sync_copy(ei_hbm.at[1, wid], idxd2)

        def gath(i, bufa, bufb, sem):
            pltpu.async_copy(t1_hbm.at[idxs2.at[i]], bufa, sem)
            pltpu.async_copy(t2_hbm.at[idxd2.at[i]], bufb, sem)

        def wait_g(bufa, bufb, sem):
            pltpu.make_async_copy(g1_hbm.at[pl.ds(0, K)], bufa, sem).wait()
            pltpu.make_async_copy(g1_hbm.at[pl.ds(0, K)], bufb, sem).wait()

        def wout(i, bufa, bufb):
            off = (base + i) * K
            c1 = pltpu.async_copy(bufa, g1_hbm.at[pl.ds(off, K)], sw0)
            c2 = pltpu.async_copy(bufb, g2_hbm.at[pl.ds(off, K)], sw0)
            c1.wait()
            c2.wait()

        gath(0, a0, b0, sg0)
        gath(1, a1, b1, sg1)

        @pl.loop(0, npair)
        def _(j):
            i = j * 2
            wait_g(a0, b0, sg0)
            wout(i, a0, b0)
            gath(i + 2, a0, b0, sg0)
            wait_g(a1, b1, sg1)
            wout(i + 1, a1, b1)

            @pl.when(j < npair - 1)
            def _():
                gath(i + 3, a1, b1, sg1)

        wait_g(a0, b0, sg0)
        wout(nchunk - 1, a0, b0)

    return k(ei_s, T1, T2)


def _edge_t_stats(e, G1, G2, wt, b, Es, eoff, block):
    D = e.shape[1]
    nb = Es // block
    ob = eoff // block

    def body(e_ref, g1_ref, g2_ref, w_ref, b_ref, st_ref, acc):
        i = pl.program_id(0)

        @pl.when(i == 0)
        def _():
            acc[...] = jnp.zeros_like(acc)

        t = (jnp.dot(e_ref[...], w_ref[...], precision=lax.Precision.HIGHEST,
                     preferred_element_type=F32)
             + b_ref[...] + _unpack_hi(g1_ref[...])
             + _unpack_hi(g2_ref[...]))
        t3 = t.reshape(block // 8, 8, D)
        acc[0] += jnp.sum(t3, axis=0)
        acc[1] += jnp.sum(t3 * t3, axis=0)

        @pl.when(i == nb - 1)
        def _():
            st_ref[...] = acc[...]

    blk = pl.BlockSpec((block, D), lambda i: (i, 0))
    eblk = pl.BlockSpec((block, D), lambda i: (ob + i, 0))
    return pl.pallas_call(
        body,
        grid=(nb,),
        in_specs=[eblk, blk, blk,
                  pl.BlockSpec((D, D), lambda i: (0, 0)),
                  pl.BlockSpec((1, D), lambda i: (0, 0))],
        out_specs=pl.BlockSpec((2, 8, D), lambda i: (0, 0, 0)),
        out_shape=jax.ShapeDtypeStruct((2, 8, D), F32),
        scratch_shapes=[pltpu.VMEM((2, 8, D), F32)],
    )(e, G1, G2, wt, b[None, :])


def _edge_sigma(e, G1, G2, wt, b, stats, gamma, beta, E_total, Es, eoff,
                block):
    D = e.shape[1]
    nb = Es // block
    ob = eoff // block
    ns = stats.shape[0]

    def body(e_ref, a2_ref, a3_ref, w_ref, b_ref, st_ref, g_ref, bt_ref,
             pf_ref, pb_ref):
        ssum = jnp.sum(st_ref[:, 0], axis=(0, 1))
        ssq = jnp.sum(st_ref[:, 1], axis=(0, 1))
        mean = ssum / E_total
        var = ssq / E_total - mean * mean
        scale = g_ref[0] * lax.rsqrt(var + 1e-5)
        shift = bt_ref[0] - mean * scale
        ew = e_ref[...]
        t = (jnp.dot(ew, w_ref[...], precision=lax.Precision.HIGHEST,
                     preferred_element_type=F32)
             + b_ref[...] + _unpack_hi(a2_ref[...])
             + _unpack_hi(a3_ref[...]))
        bn = t * scale + shift
        ee = jnp.maximum(bn, 0.0) + ew
        sg = jax.nn.sigmoid(ee)
        u2 = sg * _unpack_lo(a2_ref[...])
        u3 = sg * _unpack_lo(a3_ref[...])

        def planes(u):
            p0 = jnp.concatenate([u[:, 0:32], sg[:, 0:32],
                                  u[:, 32:64], sg[:, 32:64]], axis=1)
            p1 = jnp.concatenate([u[:, 64:96], sg[:, 64:96],
                                  u[:, 96:128], sg[:, 96:128]], axis=1)
            return jnp.stack([p0, p1], axis=0)

        pf_ref[...] = planes(u2)
        pb_ref[...] = planes(u3)

    blk = pl.BlockSpec((block, D), lambda i: (i, 0))
    eblk = pl.BlockSpec((block, D), lambda i: (ob + i, 0))
    small = pl.BlockSpec((1, D), lambda i: (0, 0))
    return pl.pallas_call(
        body,
        grid=(nb,),
        in_specs=[eblk, blk, blk,
                  pl.BlockSpec((D, D), lambda i: (0, 0)), small,
                  pl.BlockSpec((ns, 2, 8, D), lambda i: (0, 0, 0, 0)),
                  small, small],
        out_specs=[pl.BlockSpec((2, block, D), lambda i: (0, i, 0))] * 2,
        out_shape=[jax.ShapeDtypeStruct((2, Es, D), F32)] * 2,
    )(e, G1, G2, wt, b[None, :], stats, gamma[None, :], beta[None, :])


def _sc_scatter_group(ei_list, P_list, zrows, Np, idx_row):
    D = P_list[0].shape[2]
    Es = P_list[0].shape[1]
    nchunk = Es // (NW * K)
    npair = (nchunk - 1) // 2
    nrow = Np // NS
    ng = len(P_list)
    po = jax.ShapeDtypeStruct((NC, 2, Np, D), F32)

    @functools.partial(
        pl.kernel, mesh=_mesh,
        out_type=po,
        scratch_types=[
            pltpu.VMEM((nchunk, K), jnp.int32),
            pltpu.VMEM((K, D), F32),
            pltpu.VMEM((K, D), F32),
            pltpu.VMEM_SHARED((Np, D), F32),
            pltpu.SemaphoreType.DMA,
            pltpu.SemaphoreType.DMA,
            pltpu.SemaphoreType.DMA,
            pltpu.SemaphoreType.DMA,
        ],
    )
    def k(*refs):
        ei_hbms = refs[:ng]
        p_hbms = refs[ng:2 * ng]
        z_hbm = refs[2 * ng]
        out = refs[2 * ng + 1]
        idx2, f0, f1, acc, sr0, sr1, ss0, ss1 = refs[2 * ng + 2:]
        cid = lax.axis_index("c")
        sid = lax.axis_index("s")
        wid = sid * NC + cid
        base = wid * nchunk
        r0 = sid * nrow
        for p in range(2):
            pltpu.sync_copy(z_hbm, acc.at[pl.ds(r0, nrow)])
            plsc.subcore_barrier()
            for g in range(ng):
                p_hbm = p_hbms[g]
                pltpu.sync_copy(ei_hbms[g].at[idx_row, wid], idx2)

                def rd(i, buf, sem):
                    off = (base + i) * K
                    pltpu.async_copy(p_hbm.at[p, pl.ds(off, K), :], buf, sem)

                def wait_r(buf, sem):
                    pltpu.make_async_copy(
                        p_hbm.at[0, pl.ds(0, K), :], buf, sem).wait()

                def scat(i, buf):
                    pltpu.async_copy(buf, acc.at[idx2.at[i]], ss0,
                                     add=True).wait()

                rd(0, f0, sr0)
                rd(1, f1, sr1)

                @pl.loop(0, npair)
                def _(j):
                    i = j * 2
                    wait_r(f0, sr0)
                    scat(i, f0)
                    rd(i + 2, f0, sr0)
                    wait_r(f1, sr1)
                    scat(i + 1, f1)

                    @pl.when(j < npair - 1)
                    def _():
                        rd(i + 3, f1, sr1)

                wait_r(f0, sr0)
                scat(nchunk - 1, f0)

            plsc.subcore_barrier()
            pltpu.sync_copy(acc.at[pl.ds(r0, nrow)],
                            out.at[cid, p, pl.ds(r0, nrow), :])
            plsc.subcore_barrier()

    return k(*ei_list, *P_list, zrows)


def _final(A1h, h, oFs, oBs, gamma, beta, block):
    N, D = h.shape
    nb = N // block
    nparts = len(oFs)

    def body(*refs):
        a1_ref, h_ref = refs[0], refs[1]
        f_refs = refs[2:2 + nparts]
        b_refs = refs[2 + nparts:2 + 2 * nparts]
        g_ref, bt_ref, o_ref, acc, hp_ref = refs[2 + 2 * nparts:]
        p = pl.program_id(0)
        j = pl.program_id(1)

        def agg(part_refs):
            cols = []
            for c in range(4):
                pp = c // 2
                a = slice(64 * (c % 2), 64 * (c % 2) + 32)
                d = slice(64 * (c % 2) + 32, 64 * (c % 2) + 64)
                num = den = None
                for r in part_refs:
                    for cc in range(2):
                        nn = r[cc, pp, :, a]
                        dd = r[cc, pp, :, d]
                        num = nn if num is None else num + nn
                        den = dd if den is None else den + dd
                cols.append(num / (den + 1e-6))
            return jnp.concatenate(cols, axis=1)

        @pl.when(p == 0)
        def _():
            hp = a1_ref[...] + agg(f_refs) + agg(b_refs)
            hp_ref[pl.ds(j * block, block), :] = hp

            @pl.when(j == 0)
            def _():
                acc[...] = jnp.zeros_like(acc)

            h3 = hp.reshape(block // 8, 8, D)
            acc[0] += jnp.sum(h3, axis=0)
            acc[1] += jnp.sum(h3 * h3, axis=0)

        @pl.when(p == 1)
        def _():
            hp = hp_ref[pl.ds(j * block, block), :]
            mean = jnp.sum(acc[0], axis=0) / N
            var = jnp.sum(acc[1], axis=0) / N - mean * mean
            scale = g_ref[0] * lax.rsqrt(var + 1e-5)
            shift = bt_ref[0] - mean * scale
            o_ref[...] = jnp.maximum(hp * scale + shift, 0.0) + h_ref[...]

    blk = pl.BlockSpec((block, D), lambda p, j: (j, 0))
    pblk = pl.BlockSpec((2, 2, block, D),
                        lambda p, j: (0, 0, j * (1 - p), 0))
    small = pl.BlockSpec((1, D), lambda p, j: (0, 0))
    return pl.pallas_call(
        body,
        grid=(2, nb),
        in_specs=[blk, blk] + [pblk] * (2 * nparts) + [small, small],
        out_specs=blk,
        out_shape=jax.ShapeDtypeStruct((N, D), F32),
        scratch_shapes=[pltpu.VMEM((2, 8, D), F32),
                        pltpu.VMEM((N, D), F32)],
    )(A1h, h, *oFs, *oBs, gamma[None, :], beta[None, :])


def kernel(edge_index, h, e, A1_W, A1_b, A2_W, A2_b, A3_W, A3_b,
           B1_W, B1_b, B2_W, B2_b, B3_W, B3_b,
           bn_h_gamma, bn_h_beta, bn_e_gamma, bn_e_beta):
    N, D = h.shape
    E = e.shape[0]
    Es = E // S
    nchunk = Es // (NW * K)

    A1h, T1, T2 = _node_matmuls(
        h,
        (A1_W.T, B1_W.T, A2_W.T, B2_W.T, A3_W.T),
        (A1_b, B1_b, A2_b, B2_b, A3_b))

    ei6 = edge_index.reshape(2, S, NW, nchunk, K)
    ei_s = [ei6[:, s] for s in range(S)]

    G = [_sc_gather(ei_s[s], T1, T2, Es) for s in range(S)]
    stats = jnp.stack(
        [_edge_t_stats(e, G[s][0], G[s][1], B3_W.T, B3_b, Es, s * Es, 4000)
         for s in range(S)])

    P = [_edge_sigma(e, G[s][0], G[s][1], B3_W.T, B3_b, stats,
                     bn_e_gamma, bn_e_beta, E, Es, s * Es, 4000)
         for s in range(S)]

    Np = ((N + 8 * NS - 1) // (8 * NS)) * (8 * NS)
    zrows = jnp.zeros((Np // NS, D), F32)
    oFs, oBs = [], []
    for grp in ((0,), (1, 2), (3, 4)):
        eis = [ei_s[s] for s in grp]
        oFs.append(_sc_scatter_group(eis, [P[s][0] for s in grp],
                                     zrows, Np, 1))
        oBs.append(_sc_scatter_group(eis, [P[s][1] for s in grp],
                                     zrows, Np, 0))

    return _final(A1h, h, oFs, oBs, bn_h_gamma, bn_h_beta, 1000)


# speedup vs baseline: 1.1310x; 1.0002x over previous
import functools

import jax
import jax.numpy as jnp
from jax import lax
from jax.experimental import pallas as pl
from jax.experimental.pallas import tpu as pltpu
from jax.experimental.pallas import tpu_sc as plsc

F32 = jnp.float32
NC = 2
NS = 16
NW = NC * NS
K = 80
S = 5
SC2_GROUPS = ((0,), (1, 2), (3, 4))

_mesh = plsc.VectorSubcoreMesh(core_axis_name="c", subcore_axis_name="s")


def _unpack_hi(p):
    w = lax.bitcast_convert_type(p, jnp.uint32)
    return lax.bitcast_convert_type((w >> 16).astype(jnp.uint16),
                                    jnp.bfloat16).astype(F32)


def _unpack_lo(p):
    w = lax.bitcast_convert_type(p, jnp.uint32)
    return lax.bitcast_convert_type(w.astype(jnp.uint16),
                                    jnp.bfloat16).astype(F32)


def _node_matmuls(h, wts, bs, block=2000):
    N, D = h.shape
    nb = N // block

    def pack2(x, y):
        xu = lax.bitcast_convert_type(x.astype(jnp.bfloat16),
                                      jnp.uint16).astype(jnp.uint32)
        yu = lax.bitcast_convert_type(y.astype(jnp.bfloat16),
                                      jnp.uint16).astype(jnp.uint32)
        return lax.bitcast_convert_type((xu << 16) | yu, F32)

    def body(h_ref, w_ref, b_ref, oa1, ot1, ot2):
        hh = h_ref[...]
        m = [jnp.dot(hh, w_ref[i], precision=lax.Precision.HIGHEST,
                     preferred_element_type=F32) + b_ref[i] for i in range(5)]
        oa1[...] = m[0]
        ot1[...] = pack2(m[1], m[2])
        ot2[...] = pack2(m[3], m[4])

    blk = pl.BlockSpec((block, D), lambda i: (i, 0))
    return pl.pallas_call(
        body,
        grid=(nb,),
        in_specs=[blk,
                  pl.BlockSpec((5, D, D), lambda i: (0, 0, 0)),
                  pl.BlockSpec((5, 1, D), lambda i: (0, 0, 0))],
        out_specs=[blk, blk, blk],
        out_shape=[jax.ShapeDtypeStruct((N, D), F32)] * 3,
    )(h, jnp.stack(wts), jnp.stack(bs)[:, None, :])


def _sc_gather(ei_s, T1, T2, Es):
    N, D = T1.shape
    nchunk = Es // (NW * K)
    npair = (nchunk - 1) // 2
    fo = jax.ShapeDtypeStruct((Es, D), F32)

    @functools.partial(
        pl.kernel, mesh=_mesh,
        out_type=[fo, fo],
        scratch_types=[
            pltpu.VMEM((nchunk, K), jnp.int32),
            pltpu.VMEM((nchunk, K), jnp.int32),
            pltpu.VMEM((K, D), F32),
            pltpu.VMEM((K, D), F32),
            pltpu.VMEM((K, D), F32),
            pltpu.VMEM((K, D), F32),
            pltpu.SemaphoreType.DMA,
            pltpu.SemaphoreType.DMA,
            pltpu.SemaphoreType.DMA,
            pltpu.SemaphoreType.DMA,
        ],
    )
    def k(ei_hbm, t1_hbm, t2_hbm, g1_hbm, g2_hbm,
          idxs2, idxd2, a0, b0, a1, b1, sg0, sg1, sw0, sw1):
        wid = lax.axis_index("s") * NC + lax.axis_index("c")
        base = wid * nchunk
        pltpu.sync_copy(ei_hbm.at[0, wid], idxs2)
        pltpu.sync_copy(ei_hbm.at[1, wid], idxd2)

        def gath(i, bufa, bufb, sem):
            pltpu.async_copy(t1_hbm.at[idxs2.at[i]], bufa, sem)
            pltpu.async_copy(t2_hbm.at[idxd2.at[i]], bufb, sem)

        def wait_g(bufa, bufb, sem):
            pltpu.make_async_copy(g1_hbm.at[pl.ds(0, K)], bufa, sem).wait()
            pltpu.make_async_copy(g1_hbm.at[pl.ds(0, K)], bufb, sem).wait()

        def wout(i, bufa, bufb):
            off = (base + i) * K
            c1 = pltpu.async_copy(bufa, g1_hbm.at[pl.ds(off, K)], sw0)
            c2 = pltpu.async_copy(bufb, g2_hbm.at[pl.ds(off, K)], sw0)
            c1.wait()
            c2.wait()

        gath(0, a0, b0, sg0)
        gath(1, a1, b1, sg1)

        @pl.loop(0, npair)
        def _(j):
            i = j * 2
            wait_g(a0, b0, sg0)
            wout(i, a0, b0)
            gath(i + 2, a0, b0, sg0)
            wait_g(a1, b1, sg1)
            wout(i + 1, a1, b1)

            @pl.when(j < npair - 1)
            def _():
                gath(i + 3, a1, b1, sg1)

        wait_g(a0, b0, sg0)
        wout(nchunk - 1, a0, b0)

    return k(ei_s, T1, T2)


def _edge_t_stats(e, G1, G2, wt, b, Es, eoff, block):
    D = e.shape[1]
    nb = Es // block
    ob = eoff // block

    def body(e_ref, g1_ref, g2_ref, w_ref, b_ref, st_ref, acc):
        i = pl.program_id(0)

        @pl.when(i == 0)
        def _():
            acc[...] = jnp.zeros_like(acc)

        t = (jnp.dot(e_ref[...], w_ref[...], precision=lax.Precision.HIGHEST,
                     preferred_element_type=F32)
             + b_ref[...] + _unpack_hi(g1_ref[...])
             + _unpack_hi(g2_ref[...]))
        t3 = t.reshape(block // 8, 8, D)
        acc[0] += jnp.sum(t3, axis=0)
        acc[1] += jnp.sum(t3 * t3, axis=0)

        @pl.when(i == nb - 1)
        def _():
            st_ref[...] = acc[...]

    blk = pl.BlockSpec((block, D), lambda i: (i, 0))
    eblk = pl.BlockSpec((block, D), lambda i: (ob + i, 0))
    return pl.pallas_call(
        body,
        grid=(nb,),
        in_specs=[eblk, blk, blk,
                  pl.BlockSpec((D, D), lambda i: (0, 0)),
                  pl.BlockSpec((1, D), lambda i: (0, 0))],
        out_specs=pl.BlockSpec((2, 8, D), lambda i: (0, 0, 0)),
        out_shape=jax.ShapeDtypeStruct((2, 8, D), F32),
        scratch_shapes=[pltpu.VMEM((2, 8, D), F32)],
    )(e, G1, G2, wt, b[None, :])


def _edge_sigma(e, G1, G2, wt, b, stats, gamma, beta, E_total, Es, eoff,
                block):
    D = e.shape[1]
    nb = Es // block
    ob = eoff // block
    ns = stats.shape[0]

    def body(e_ref, a2_ref, a3_ref, w_ref, b_ref, st_ref, g_ref, bt_ref,
             pf_ref, pb_ref):
        ssum = jnp.sum(st_ref[:, 0], axis=(0, 1))
        ssq = jnp.sum(st_ref[:, 1], axis=(0, 1))
        mean = ssum / E_total
        var = ssq / E_total - mean * mean
        scale = g_ref[0] * lax.rsqrt(var + 1e-5)
        shift = bt_ref[0] - mean * scale
        ew = e_ref[...]
        t = (jnp.dot(ew, w_ref[...], precision=lax.Precision.HIGHEST,
                     preferred_element_type=F32)
             + b_ref[...] + _unpack_hi(a2_ref[...])
             + _unpack_hi(a3_ref[...]))
        bn = t * scale + shift
        ee = jnp.maximum(bn, 0.0) + ew
        sg = jax.nn.sigmoid(ee)
        u2 = sg * _unpack_lo(a2_ref[...])
        u3 = sg * _unpack_lo(a3_ref[...])

        def planes(u):
            p0 = jnp.concatenate([u[:, 0:32], sg[:, 0:32],
                                  u[:, 32:64], sg[:, 32:64]], axis=1)
            p1 = jnp.concatenate([u[:, 64:96], sg[:, 64:96],
                                  u[:, 96:128], sg[:, 96:128]], axis=1)
            return jnp.stack([p0, p1], axis=0)

        pf_ref[...] = planes(u2)
        pb_ref[...] = planes(u3)

    blk = pl.BlockSpec((block, D), lambda i: (i, 0))
    eblk = pl.BlockSpec((block, D), lambda i: (ob + i, 0))
    small = pl.BlockSpec((1, D), lambda i: (0, 0))
    return pl.pallas_call(
        body,
        grid=(nb,),
        in_specs=[eblk, blk, blk,
                  pl.BlockSpec((D, D), lambda i: (0, 0)), small,
                  pl.BlockSpec((ns, 2, 8, D), lambda i: (0, 0, 0, 0)),
                  small, small],
        out_specs=[pl.BlockSpec((2, block, D), lambda i: (0, i, 0))] * 2,
        out_shape=[jax.ShapeDtypeStruct((2, Es, D), F32)] * 2,
    )(e, G1, G2, wt, b[None, :], stats, gamma[None, :], beta[None, :])


def _sc_scatter_group(ei_list, P_list, zrows, Np, idx_row):
    D = P_list[0].shape[2]
    Es = P_list[0].shape[1]
    nchunk = Es // (NW * K)
    npair = (nchunk - 1) // 2
    nrow = Np // NS
    ng = len(P_list)
    po = jax.ShapeDtypeStruct((NC, 2, Np, D), F32)

    @functools.partial(
        pl.kernel, mesh=_mesh,
        out_type=po,
        scratch_types=[
            pltpu.VMEM((nchunk, K), jnp.int32),
            pltpu.VMEM((K, D), F32),
            pltpu.VMEM((K, D), F32),
            pltpu.VMEM_SHARED((Np, D), F32),
            pltpu.SemaphoreType.DMA,
            pltpu.SemaphoreType.DMA,
            pltpu.SemaphoreType.DMA,
            pltpu.SemaphoreType.DMA,
        ],
    )
    def k(*refs):
        ei_hbms = refs[:ng]
        p_hbms = refs[ng:2 * ng]
        z_hbm = refs[2 * ng]
        out = refs[2 * ng + 1]
        idx2, f0, f1, acc, sr0, sr1, ss0, ss1 = refs[2 * ng + 2:]
        cid = lax.axis_index("c")
        sid = lax.axis_index("s")
        wid = sid * NC + cid
        base = wid * nchunk
        r0 = sid * nrow
        for p in range(2):
            pltpu.sync_copy(z_hbm, acc.at[pl.ds(r0, nrow)])
            plsc.subcore_barrier()
            for g in range(ng):
                p_hbm = p_hbms[g]
                pltpu.sync_copy(ei_hbms[g].at[idx_row, wid], idx2)

                def rd(i, buf, sem):
                    off = (base + i) * K
                    pltpu.async_copy(p_hbm.at[p, pl.ds(off, K), :], buf, sem)

                def wait_r(buf, sem):
                    pltpu.make_async_copy(
                        p_hbm.at[0, pl.ds(0, K), :], buf, sem).wait()

                def scat(i, buf):
                    pltpu.async_copy(buf, acc.at[idx2.at[i]], ss0,
                                     add=True).wait()

                rd(0, f0, sr0)
                rd(1, f1, sr1)

                @pl.loop(0, npair)
                def _(j):
                    i = j * 2
                    wait_r(f0, sr0)
                    scat(i, f0)
                    rd(i + 2, f0, sr0)
                    wait_r(f1, sr1)
                    scat(i + 1, f1)

                    @pl.when(j < npair - 1)
                    def _():
                        rd(i + 3, f1, sr1)

                wait_r(f0, sr0)
                scat(nchunk - 1, f0)

            plsc.subcore_barrier()
            pltpu.sync_copy(acc.at[pl.ds(r0, nrow)],
                            out.at[cid, p, pl.ds(r0, nrow), :])
            plsc.subcore_barrier()

    return k(*ei_list, *P_list, zrows)


def _final(A1h, h, oFs, oBs, gamma, beta, block):
    N, D = h.shape
    nb = N // block
    nparts = len(oFs)

    def body(*refs):
        a1_ref, h_ref = refs[0], refs[1]
        f_refs = refs[2:2 + nparts]
        b_refs = refs[2 + nparts:2 + 2 * nparts]
        g_ref, bt_ref, o_ref, acc, hp_ref = refs[2 + 2 * nparts:]
        p = pl.program_id(0)
        j = pl.program_id(1)

        def agg(part_refs):
            cols = []
            for c in range(4):
                pp = c // 2
                a = slice(64 * (c % 2), 64 * (c % 2) + 32)
                d = slice(64 * (c % 2) + 32, 64 * (c % 2) + 64)
                num = den = None
                for r in part_refs:
                    for cc in range(2):
                        nn = r[cc, pp, :, a]
                        dd = r[cc, pp, :, d]
                        num = nn if num is None else num + nn
                        den = dd if den is None else den + dd
                cols.append(num / (den + 1e-6))
            return jnp.concatenate(cols, axis=1)

        @pl.when(p == 0)
        def _():
            hp = a1_ref[...] + agg(f_refs) + agg(b_refs)
            hp_ref[pl.ds(j * block, block), :] = hp

            @pl.when(j == 0)
            def _():
                acc[...] = jnp.zeros_like(acc)

            h3 = hp.reshape(block // 8, 8, D)
            acc[0] += jnp.sum(h3, axis=0)
            acc[1] += jnp.sum(h3 * h3, axis=0)

        @pl.when(p == 1)
        def _():
            hp = hp_ref[pl.ds(j * block, block), :]
            mean = jnp.sum(acc[0], axis=0) / N
            var = jnp.sum(acc[1], axis=0) / N - mean * mean
            scale = g_ref[0] * lax.rsqrt(var + 1e-5)
            shift = bt_ref[0] - mean * scale
            o_ref[...] = jnp.maximum(hp * scale + shift, 0.0) + h_ref[...]

    blk = pl.BlockSpec((block, D), lambda p, j: (j, 0))
    pblk = pl.BlockSpec((2, 2, block, D),
                        lambda p, j: (0, 0, j * (1 - p), 0))
    small = pl.BlockSpec((1, D), lambda p, j: (0, 0))
    return pl.pallas_call(
        body,
        grid=(2, nb),
        in_specs=[blk, blk] + [pblk] * (2 * nparts) + [small, small],
        out_specs=blk,
        out_shape=jax.ShapeDtypeStruct((N, D), F32),
        scratch_shapes=[pltpu.VMEM((2, 8, D), F32),
                        pltpu.VMEM((N, D), F32)],
    )(A1h, h, *oFs, *oBs, gamma[None, :], beta[None, :])


def kernel(edge_index, h, e, A1_W, A1_b, A2_W, A2_b, A3_W, A3_b,
           B1_W, B1_b, B2_W, B2_b, B3_W, B3_b,
           bn_h_gamma, bn_h_beta, bn_e_gamma, bn_e_beta):
    N, D = h.shape
    E = e.shape[0]
    Es = E // S
    nchunk = Es // (NW * K)

    A1h, T1, T2 = _node_matmuls(
        h,
        (A1_W.T, B1_W.T, A2_W.T, B2_W.T, A3_W.T),
        (A1_b, B1_b, A2_b, B2_b, A3_b))

    ei6 = edge_index.reshape(2, S, NW, nchunk, K)
    ei_s = [ei6[:, s] for s in range(S)]

    G = [_sc_gather(ei_s[s], T1, T2, Es) for s in range(S)]
    stats = jnp.stack(
        [_edge_t_stats(e, G[s][0], G[s][1], B3_W.T, B3_b, Es, s * Es, 4000)
         for s in range(S)])

    P = [_edge_sigma(e, G[s][0], G[s][1], B3_W.T, B3_b, stats,
                     bn_e_gamma, bn_e_beta, E, Es, s * Es, 4000)
         for s in range(S)]

    Np = ((N + 8 * NS - 1) // (8 * NS)) * (8 * NS)
    zrows = jnp.zeros((Np // NS, D), F32)
    oFs, oBs = [], []
    for grp in SC2_GROUPS:
        eis = [ei_s[s] for s in grp]
        oFs.append(_sc_scatter_group(eis, [P[s][0] for s in grp],
                                     zrows, Np, 1))
        oBs.append(_sc_scatter_group(eis, [P[s][1] for s in grp],
                                     zrows, Np, 0))

    return _final(A1h, h, oFs, oBs, bn_h_gamma, bn_h_beta, 1000)
